# Initial kernel scaffold; baseline (speedup 1.0000x reference)
import jax
import jax.numpy as jnp
from jax.experimental import pallas as pl


def kernel(p, p_gs, f, group_idx, gs_group_idx, W_sa, W_pe, W_sa_gs, W_pe_gs, alpha, W1, W2, W_pm, W_gate, gamma, beta, W_post):
    raise NotImplementedError("write your pallas kernel here")


# speedup vs baseline: 6.2817x; 6.2817x over previous
import functools

import jax
import jax.numpy as jnp
from jax import lax
from jax.experimental import pallas as pl
from jax.experimental.pallas import tpu as pltpu
from jax.experimental.pallas import tpu_sc as plsc

N = 50000
K = 16
C = 64
CH = 32
NW = 32
NPAD = 50176
PTS = NPAD // NW
NCHUNK = PTS // CH
BLK = 3584
GRID = NPAD // BLK


def _make_gather_max():
  info = plsc.get_sparse_core_info()
  nc = info.num_cores
  mesh = plsc.VectorSubcoreMesh(core_axis_name="c", subcore_axis_name="s")
  ngath = CH * K // 128

  @functools.partial(
      pl.kernel,
      mesh=mesh,
      compiler_params=pltpu.CompilerParams(use_tc_tiling_on_sc=False),
      out_type=jax.ShapeDtypeStruct((NPAD, C), jnp.float32),
      scratch_types=[
          pltpu.VMEM((CH * K,), jnp.int32),
          pltpu.VMEM((CH * K, C), jnp.float32),
          pltpu.VMEM((CH, C), jnp.float32),
          pltpu.SemaphoreType.DMA,
      ],
  )
  def gather_max(table, idxf, out, idx_v, rows_v, out_v, sem):
    wid = lax.axis_index("s") * nc + lax.axis_index("c")

    def chunk(t, carry):
      p0 = wid * PTS + t * CH
      pltpu.sync_copy(idxf.at[pl.ds(p0 * K, CH * K)], idx_v)
      cps = [
          pltpu.async_copy(
              table.at[idx_v.at[pl.ds(j * 128, 128)]],
              rows_v.at[pl.ds(j * 128, 128)],
              sem,
          )
          for j in range(ngath)
      ]
      for cp in cps:
        cp.wait()

      def point(jj, cc):
        r = jj * K
        for cblk in range(C // 16):
          sl = pl.ds(cblk * 16, 16)
          acc = rows_v[r, sl]
          for kk in range(1, K):
            acc = jnp.maximum(acc, rows_v[r + kk, sl])
          out_v[jj, sl] = acc
        return cc

      lax.fori_loop(0, CH, point, 0)
      pltpu.sync_copy(out_v, out.at[pl.ds(p0, CH)])
      return carry

    lax.fori_loop(0, NCHUNK, chunk, 0)

  return gather_max


def _cols_matmul(x, w, ncols):
  acc = x[:, 0:1] * w[0:1, :]
  for c in range(1, ncols):
    acc = acc + x[:, c : c + 1] * w[c : c + 1, :]
  return acc


def _pre_body(al, p_ref, pg_ref, f_ref, wpe, wpeg, wsa, wsag,
              u1_ref, u2_ref, base_ref):
  a = jax.nn.sigmoid(al[0, 0])
  pe1 = _cols_matmul(p_ref[...] * 40.0, wpe[...], 3)
  pe2 = _cols_matmul(pg_ref[...], wpeg[...], 3)
  h1 = _cols_matmul(f_ref[...], wsa[...], 4)
  h2 = _cols_matmul(f_ref[...], wsag[...], 4)
  u1_ref[...] = (h1 + pe1) * a
  u2_ref[...] = (h2 + pe2) * (1.0 - a)
  base_ref[...] = -(a * pe1 + (1.0 - a) * pe2)


def _blend_body(g1_ref, g2_ref, b_ref, o_ref):
  o_ref[...] = g1_ref[...] + g2_ref[...] + b_ref[...]


def _res_body(fl_ref, agg_ref, w1_ref, w2_ref, o_ref):
  h = jax.nn.gelu(
      jnp.dot(agg_ref[...], w1_ref[...], preferred_element_type=jnp.float32))
  o_ref[...] = fl_ref[...] + jnp.dot(
      h, w2_ref[...], preferred_element_type=jnp.float32)


def _final_body(fl_ref, wpm, wg, gam, bet, wpost, o_ref):
  x = fl_ref[...]
  g = jnp.dot(x, wpm[...], preferred_element_type=jnp.float32)
  gate = jax.nn.sigmoid(
      jnp.dot(x, wg[...], preferred_element_type=jnp.float32))
  fo = (g * gate + x) * gam[...] + bet[...]
  o_ref[...] = jnp.dot(fo, wpost[...], preferred_element_type=jnp.float32)


def _row_spec(cols):
  return pl.BlockSpec((BLK, cols), lambda i: (i, 0))


def _full_spec(r, c):
  return pl.BlockSpec((r, c), lambda i: (0, 0))


def kernel(p, p_gs, f, group_idx, gs_group_idx, W_sa, W_pe, W_sa_gs, W_pe_gs,
           alpha, W1, W2, W_pm, W_gate, gamma, beta, W_post):
  pad = NPAD - N
  p_ = jnp.pad(p, ((0, pad), (0, 0)))
  pg_ = jnp.pad(p_gs, ((0, pad), (0, 0)))
  f_ = jnp.pad(f, ((0, pad), (0, 0)))
  idx1 = jnp.pad(group_idx.astype(jnp.int32).reshape(-1), (0, pad * K))
  idx2 = jnp.pad(gs_group_idx.astype(jnp.int32).reshape(-1), (0, pad * K))
  al2 = alpha.reshape(1, 1)

  u1, u2, base = pl.pallas_call(
      _pre_body,
      grid=(GRID,),
      in_specs=[
          _full_spec(1, 1),
          _row_spec(3), _row_spec(3), _row_spec(4),
          _full_spec(3, C), _full_spec(3, C),
          _full_spec(4, C), _full_spec(4, C),
      ],
      out_specs=[_row_spec(C)] * 3,
      out_shape=[jax.ShapeDtypeStruct((NPAD, C), jnp.float32)] * 3,
  )(al2, p_, pg_, f_, W_pe, W_pe_gs, W_sa, W_sa_gs)

  gather_max = _make_gather_max()
  g1 = gather_max(u1, idx1)
  g2 = gather_max(u2, idx2)

  f_local = pl.pallas_call(
      _blend_body,
      grid=(GRID,),
      in_specs=[_row_spec(C)] * 3,
      out_specs=_row_spec(C),
      out_shape=jax.ShapeDtypeStruct((NPAD, C), jnp.float32),
  )(g1, g2, base)

  nres, hid = W1.shape[0], W1.shape[2]
  for i in range(nres):
    agg = gather_max(f_local, idx1)
    f_local = pl.pallas_call(
        _res_body,
        grid=(GRID,),
        in_specs=[_row_spec(C), _row_spec(C),
                  _full_spec(C, hid), _full_spec(hid, C)],
        out_specs=_row_spec(C),
        out_shape=jax.ShapeDtypeStruct((NPAD, C), jnp.float32),
    )(f_local, agg, W1[i], W2[i])

  ch = W_post.shape[1]
  out = pl.pallas_call(
      _final_body,
      grid=(GRID,),
      in_specs=[_row_spec(C), _full_spec(C, C), _full_spec(C, C),
                _full_spec(1, C), _full_spec(1, C), _full_spec(C, ch)],
      out_specs=_row_spec(ch),
      out_shape=jax.ShapeDtypeStruct((NPAD, ch), jnp.float32),
  )(f_local, W_pm, W_gate, gamma.reshape(1, C), beta.reshape(1, C), W_post)

  return out[:N]
